# Initial kernel scaffold; baseline (speedup 1.0000x reference)
import jax
import jax.numpy as jnp
from jax.experimental import pallas as pl


def kernel(x, e, edge_attr, W_comb, b_comb, Wq, bq, Wk, bk, Wv, bv, Wo, bo):
    raise NotImplementedError("write your pallas kernel here")


# speedup vs baseline: 4.0403x; 4.0403x over previous
import functools

import jax
import jax.numpy as jnp
from jax.experimental import pallas as pl

HEADS = 8
BQ = 128
NEG = float(-1e30)


def _gelu(v):
    return 0.5 * v * (1.0 + jax.lax.erf(v * (2.0 ** -0.5)))


def _qkv_kernel(xa_ref, xb_ref, ea_ref, wca_ref, wcb_ref, wce_ref, bc_ref,
                wq_ref, bqv_ref, wk_ref, bk_ref, wv_ref, bv_ref,
                q_ref, k_ref, v_ref):
    acc = jnp.dot(xa_ref[...], wca_ref[...], preferred_element_type=jnp.float32)
    acc += jnp.dot(xb_ref[...], wcb_ref[...], preferred_element_type=jnp.float32)
    acc += jnp.dot(ea_ref[...], wce_ref[...], preferred_element_type=jnp.float32)
    ex = _gelu(acc + bc_ref[...])
    q_ref[...] = jnp.dot(ex, wq_ref[...], preferred_element_type=jnp.float32) + bqv_ref[...]
    k_ref[...] = jnp.dot(ex, wk_ref[...], preferred_element_type=jnp.float32) + bk_ref[...]
    v_ref[...] = jnp.dot(ex, wv_ref[...], preferred_element_type=jnp.float32) + bv_ref[...]


def _attn_kernel(nblk, dh, qc_ref, kp_ref, kc_ref, kn_ref, vp_ref, vc_ref, vn_ref,
                 gq_ref, gp_ref, gc_ref, gn_ref, wo_ref, bo_ref,
                 h0_ref, sc_ref, st_ref):
    i = pl.program_id(0)
    kw = jnp.concatenate([kp_ref[...], kc_ref[...], kn_ref[...]], axis=0)
    vw = jnp.concatenate([vp_ref[...], vc_ref[...], vn_ref[...]], axis=0)
    grow = jnp.concatenate([gp_ref[...], gc_ref[...], gn_ref[...]], axis=1)
    gq = gq_ref[...]
    col = jax.lax.broadcasted_iota(jnp.int32, (BQ, 3 * BQ), 1)
    valid = ((col >= BQ) | (i > 0)) & ((col < 2 * BQ) | (i < nblk - 1))
    mask = (gq == grow) & valid
    scale = float(dh) ** -0.5
    outs = []
    for h in range(HEADS):
        qh = qc_ref[:, h * dh:(h + 1) * dh] * scale
        kh = kw[:, h * dh:(h + 1) * dh]
        vh = vw[:, h * dh:(h + 1) * dh]
        lg = jax.lax.dot_general(qh, kh, (((1,), (1,)), ((), ())),
                                 preferred_element_type=jnp.float32)
        lg = jnp.where(mask, lg, NEG)
        m = jnp.max(lg, axis=1, keepdims=True)
        p = jnp.exp(lg - m)
        attn = p / jnp.sum(p, axis=1, keepdims=True)
        outs.append(jnp.dot(attn, vh, preferred_element_type=jnp.float32))
    o = jnp.concatenate(outs, axis=1)
    o = jnp.dot(o, wo_ref[...], preferred_element_type=jnp.float32) + bo_ref[...]
    hh = _gelu(o)
    half = dh // 2
    h0s, ss = [], []
    for h in range(HEADS):
        h0s.append(hh[:, h * dh:h * dh + half])
        ss.append(jnp.mean(hh[:, h * dh + half:(h + 1) * dh], axis=1, keepdims=True))
    s = jnp.concatenate(ss, axis=1)
    h0_ref[...] = jnp.concatenate(h0s, axis=1)
    sc_ref[...] = s
    st_ref[...] = s.T


def _alpha_kernel(nblk, half, h0_ref, sq_ref, stp_ref, stc_ref, stn_ref,
                  gq_ref, gp_ref, gc_ref, gn_ref, w_ref):
    i = pl.program_id(0)
    sw = jnp.concatenate([stp_ref[...], stc_ref[...], stn_ref[...]], axis=1)
    grow = jnp.concatenate([gp_ref[...], gc_ref[...], gn_ref[...]], axis=1)
    gq = gq_ref[...]
    col = jax.lax.broadcasted_iota(jnp.int32, (BQ, 3 * BQ), 1)
    valid = ((col >= BQ) | (i > 0)) & ((col < 2 * BQ) | (i < nblk - 1))
    mask = (gq == grow) & valid
    sq = sq_ref[...]
    parts = []
    for h in range(HEADS):
        swh = jnp.broadcast_to(sw[h:h + 1, :], (BQ, 3 * BQ))
        lg = jnp.where(mask, swh, NEG)
        m = jnp.max(lg, axis=1, keepdims=True)
        denom = jnp.sum(jnp.exp(lg - m), axis=1, keepdims=True)
        alpha = jnp.exp(sq[:, h:h + 1] - m) / (denom + 1e-16)
        parts.append(h0_ref[:, h * half:(h + 1) * half] * alpha)
    w_ref[...] = jnp.concatenate(parts, axis=1)


def kernel(x, e, edge_attr, W_comb, b_comb, Wq, bq, Wk, bk, Wv, bv, Wo, bo):
    n, din = x.shape
    m = e.shape[1]
    edim = edge_attr.shape[1]
    dout = Wq.shape[0]
    dh = dout // HEADS
    half = dh // 2

    order = jnp.argsort(e[0])
    e0s = jnp.take(e[0], order).astype(jnp.int32)
    e1s = jnp.take(e[1], order).astype(jnp.int32)
    xa = jnp.take(x, e0s, axis=0)
    xb = jnp.take(x, e1s, axis=0)
    ea = jnp.take(edge_attr, order, axis=0)

    ba = 1280 if m % 1280 == 0 else BQ
    f32 = jnp.float32

    q, k, v = pl.pallas_call(
        _qkv_kernel,
        grid=(m // ba,),
        in_specs=[
            pl.BlockSpec((ba, din), lambda i: (i, 0)),
            pl.BlockSpec((ba, din), lambda i: (i, 0)),
            pl.BlockSpec((ba, edim), lambda i: (i, 0)),
            pl.BlockSpec((din, dout), lambda i: (0, 0)),
            pl.BlockSpec((din, dout), lambda i: (0, 0)),
            pl.BlockSpec((edim, dout), lambda i: (0, 0)),
            pl.BlockSpec((1, dout), lambda i: (0, 0)),
            pl.BlockSpec((dout, dout), lambda i: (0, 0)),
            pl.BlockSpec((1, dout), lambda i: (0, 0)),
            pl.BlockSpec((dout, dout), lambda i: (0, 0)),
            pl.BlockSpec((1, dout), lambda i: (0, 0)),
            pl.BlockSpec((dout, dout), lambda i: (0, 0)),
            pl.BlockSpec((1, dout), lambda i: (0, 0)),
        ],
        out_specs=[pl.BlockSpec((ba, dout), lambda i: (i, 0))] * 3,
        out_shape=[jax.ShapeDtypeStruct((m, dout), f32)] * 3,
    )(xa, xb, ea,
      W_comb[:, :din].T, W_comb[:, din:2 * din].T, W_comb[:, 2 * din:].T,
      b_comb.reshape(1, dout),
      Wq.T, bq.reshape(1, dout), Wk.T, bk.reshape(1, dout),
      Wv.T, bv.reshape(1, dout))

    nblk = m // BQ
    g_col = e0s.reshape(m, 1)
    g_row = e0s.reshape(1, m)

    cen = lambda i: (i, 0)
    prv = lambda i: (jnp.maximum(i - 1, 0), 0)
    nxt = lambda i: (jnp.minimum(i + 1, nblk - 1), 0)
    rcen = lambda i: (0, i)
    rprv = lambda i: (0, jnp.maximum(i - 1, 0))
    rnxt = lambda i: (0, jnp.minimum(i + 1, nblk - 1))

    h0, s_col, s_t = pl.pallas_call(
        functools.partial(_attn_kernel, nblk, dh),
        grid=(nblk,),
        in_specs=[
            pl.BlockSpec((BQ, dout), cen),
            pl.BlockSpec((BQ, dout), prv),
            pl.BlockSpec((BQ, dout), cen),
            pl.BlockSpec((BQ, dout), nxt),
            pl.BlockSpec((BQ, dout), prv),
            pl.BlockSpec((BQ, dout), cen),
            pl.BlockSpec((BQ, dout), nxt),
            pl.BlockSpec((BQ, 1), lambda i: (i, 0)),
            pl.BlockSpec((1, BQ), rprv),
            pl.BlockSpec((1, BQ), rcen),
            pl.BlockSpec((1, BQ), rnxt),
            pl.BlockSpec((dout, dout), lambda i: (0, 0)),
            pl.BlockSpec((1, dout), lambda i: (0, 0)),
        ],
        out_specs=[
            pl.BlockSpec((BQ, dout // 2), lambda i: (i, 0)),
            pl.BlockSpec((BQ, HEADS), lambda i: (i, 0)),
            pl.BlockSpec((HEADS, BQ), lambda i: (0, i)),
        ],
        out_shape=[
            jax.ShapeDtypeStruct((m, dout // 2), f32),
            jax.ShapeDtypeStruct((m, HEADS), f32),
            jax.ShapeDtypeStruct((HEADS, m), f32),
        ],
    )(q, k, k, k, v, v, v, g_col, g_row, g_row, g_row, Wo.T, bo.reshape(1, dout))

    w = pl.pallas_call(
        functools.partial(_alpha_kernel, nblk, half),
        grid=(nblk,),
        in_specs=[
            pl.BlockSpec((BQ, dout // 2), lambda i: (i, 0)),
            pl.BlockSpec((BQ, HEADS), lambda i: (i, 0)),
            pl.BlockSpec((HEADS, BQ), rprv),
            pl.BlockSpec((HEADS, BQ), rcen),
            pl.BlockSpec((HEADS, BQ), rnxt),
            pl.BlockSpec((BQ, 1), lambda i: (i, 0)),
            pl.BlockSpec((1, BQ), rprv),
            pl.BlockSpec((1, BQ), rcen),
            pl.BlockSpec((1, BQ), rnxt),
        ],
        out_specs=pl.BlockSpec((BQ, dout // 2), lambda i: (i, 0)),
        out_shape=jax.ShapeDtypeStruct((m, dout // 2), f32),
    )(h0, s_col, s_t, s_t, s_t, g_col, g_row, g_row, g_row)

    out = jax.ops.segment_sum(w, e1s, num_segments=n)
    return out
